# baseline (speedup 1.0000x reference)
import functools

import jax
import jax.numpy as jnp
from jax import lax
from jax.experimental import pallas as pl
from jax.experimental.pallas import tpu as pltpu
from jax.experimental.pallas import tpu_sc as plsc

_V = 1024
_D = 3
_N = 16384

_NC = 2
_NS = 16
_L = 16
_NW = _NC * _NS
_BPW = _N // _NW
_TB = 128


def _gather_call(table, idx):
  mesh = plsc.VectorSubcoreMesh(core_axis_name="c", subcore_axis_name="s")

  @functools.partial(
      pl.kernel,
      mesh=mesh,
      out_type=jax.ShapeDtypeStruct((_N, _D), jnp.float32),
      compiler_params=pltpu.CompilerParams(needs_layout_passes=False),
      scratch_types=[
          pltpu.VMEM((_TB, _D), jnp.float32),
          pltpu.VMEM((_V * _D,), jnp.float32),
          pltpu.VMEM((_BPW,), jnp.int32),
          pltpu.VMEM((_BPW, _D), jnp.float32),
          pltpu.SemaphoreType.DMA,
      ],
  )
  def k(table_hbm, idx_hbm, out_hbm, stage_v, tab_v, idx_v, out_v, sem_i):
    wid = lax.axis_index("s") * _NC + lax.axis_index("c")
    base = wid * _BPW
    cp_i = pltpu.async_copy(idx_hbm.at[pl.ds(base, _BPW)], idx_v, sem_i)
    loc = lax.iota(jnp.int32, _L)
    for ch in range(_V // _TB):
      pltpu.sync_copy(table_hbm.at[pl.ds(ch * _TB, _TB)], stage_v)
      for j in range(_TB // _L):
        r16 = loc + j * _L
        for c in range(_D):
          cc = jnp.full((_L,), c, jnp.int32)
          v = plsc.load_gather(stage_v, [r16, cc])
          tab_v[pl.ds(c * _V + ch * _TB + j * _L, _L)] = v
    cp_i.wait()
    for j in range(_BPW // _L):
      rows = idx_v[pl.ds(j * _L, _L)]
      pos = loc + j * _L
      for c in range(_D):
        cc = jnp.full((_L,), c, jnp.int32)
        col = plsc.load_gather(tab_v, [rows + c * _V])
        plsc.store_scatter(out_v, [pos, cc], col)
    pltpu.sync_copy(out_v, out_hbm.at[pl.ds(base, _BPW)])

  return k(table, idx)


def kernel(embedding, indices):
  return _gather_call(embedding, indices)


# speedup vs baseline: 1.2972x; 1.2972x over previous
import functools

import jax
import jax.numpy as jnp
from jax import lax
from jax.experimental import pallas as pl
from jax.experimental.pallas import tpu as pltpu
from jax.experimental.pallas import tpu_sc as plsc

_V = 1024
_D = 3
_N = 16384

_NC = 2
_NS = 16
_L = 16
_NW = _NC * _NS
_BPW = _N // _NW


def _gather_call(table, idx):
  mesh = plsc.VectorSubcoreMesh(core_axis_name="c", subcore_axis_name="s")

  @functools.partial(
      pl.kernel,
      mesh=mesh,
      out_type=jax.ShapeDtypeStruct((_N * _D,), jnp.float32),
      compiler_params=pltpu.CompilerParams(
          needs_layout_passes=False, skip_device_barrier=True),
      scratch_types=[
          pltpu.VMEM((_V * _D,), jnp.float32),
          pltpu.VMEM((_BPW,), jnp.int32),
          pltpu.VMEM((_BPW * _D,), jnp.float32),
          pltpu.SemaphoreType.DMA,
          pltpu.SemaphoreType.DMA,
      ],
  )
  def k(table_hbm, idx_hbm, out_hbm, tab_v, idx_v, out_v, sem_t, sem_i):
    wid = lax.axis_index("s") * _NC + lax.axis_index("c")
    base = wid * _BPW
    cp_t = pltpu.async_copy(table_hbm, tab_v, sem_t)
    cp_i = pltpu.async_copy(idx_hbm.at[pl.ds(base, _BPW)], idx_v, sem_i)
    cp_t.wait()
    cp_i.wait()
    iota16 = lax.iota(jnp.int32, _L)
    for j in range(_BPW // _L):
      rows = idx_v[pl.ds(j * _L, _L)]
      offs = rows * _D
      pos = (iota16 + (j * _L)) * _D
      for c in range(_D):
        col = plsc.load_gather(tab_v, [offs + c])
        plsc.store_scatter(out_v, [pos + c], col)
    pltpu.sync_copy(out_v, out_hbm.at[pl.ds(base * _D, _BPW * _D)])

  return k(table, idx)


def kernel(embedding, indices):
  flat = _gather_call(embedding.reshape(_V * _D), indices)
  return flat.reshape(_N, _D)
